# baseline (speedup 1.0000x reference)
import functools

import jax
import jax.numpy as jnp
import numpy as np
from jax import lax
from jax.experimental import pallas as pl
from jax.experimental.pallas import tpu as pltpu
from jax.experimental.pallas import tpu_sc as plsc

EPS = 1e-5

NC = 2
NS = 16
L = 16
NW = NC * NS

K = 80


def _col_perm(d):
    p = []
    for j in range(0, d, 2 * L):
        p.extend(j + 2 * np.arange(L))
        p.extend(j + 2 * np.arange(L) + 1)
    return np.asarray(p, np.int32)


def _sc_spmm_partials(feat, src, dst, w, zeros):
    n, d = feat.shape
    e = src.shape[0]
    ept = e // NW
    nchunk = ept // K

    rpt = (n // NS) // 8 * 8
    rem = n - NS * rpt
    rem_base = NS * rpt

    mesh = plsc.VectorSubcoreMesh(core_axis_name="c", subcore_axis_name="s")

    @functools.partial(
        pl.kernel,
        mesh=mesh,
        out_type=jax.ShapeDtypeStruct((NC, n, d), jnp.float32),
        scratch_types=[
            pltpu.VMEM((ept,), jnp.int32),
            pltpu.VMEM((ept + L,), jnp.float32),
            pltpu.VMEM((K,), jnp.int32),
            pltpu.VMEM((K,), jnp.int32),
            pltpu.VMEM((K, d), jnp.float32),
            pltpu.VMEM((K, d), jnp.float32),
            pltpu.VMEM_SHARED((n, d), jnp.float32),
            pltpu.SemaphoreType.DMA,
            pltpu.SemaphoreType.DMA,
            pltpu.SemaphoreType.DMA,
            pltpu.SemaphoreType.DMA,
            pltpu.SemaphoreType.DMA,
            pltpu.SemaphoreType.DMA,
        ],
    )
    def k(feat_h, src_h, dst_h, w_h, zeros_h, out_h,
          src_v, w_v, d0, d1, rbf0, rbf1, acc,
          gsem0, gsem1, dsem0, dsem1, ssem0, ssem1):
        c = lax.axis_index("c")
        s = lax.axis_index("s")
        wid = s * NC + c
        rbf = (rbf0, rbf1)
        dstb = (d0, d1)
        gsem = (gsem0, gsem1)
        dsem = (dsem0, dsem1)
        ssem = (ssem0, ssem1)

        def issue_stage(t, b):
            pltpu.async_copy(dst_h.at[pl.ds(wid * ept + t * K, K)],
                             dstb[b], dsem[b])
            pltpu.async_copy(feat_h.at[src_v.at[pl.ds(t * K, K)]],
                             rbf[b], gsem[b])

        def wait_gather(b):
            pltpu.make_async_copy(feat_h.at[pl.ds(0, K)], rbf[b],
                                  gsem[b]).wait()

        def wait_dst(b):
            pltpu.make_async_copy(dst_h.at[pl.ds(0, K)], dstb[b],
                                  dsem[b]).wait()

        def wait_scatter(b):
            pltpu.make_async_copy(zeros_h.at[pl.ds(0, K)], rbf[b],
                                  ssem[b]).wait()

        def issue_scatter(b):
            pltpu.async_copy(rbf[b], acc.at[dstb[b]], ssem[b], add=True)

        def scale_rows(t, b):
            for g in range(-(-K // L)):
                wvec = w_v[pl.ds(t * K + g * L, L)]
                for ii in range(min(L, K - g * L)):
                    i = g * L + ii
                    ws = wvec[ii]
                    for j in range(d // L):
                        rbf[b][i, pl.ds(j * L, L)] = (
                            rbf[b][i, pl.ds(j * L, L)] * ws)

        pltpu.sync_copy(src_h.at[pl.ds(wid * ept, ept)], src_v)
        pltpu.sync_copy(w_h.at[pl.ds(wid * ept, ept + L)], w_v)

        issue_stage(0, 0)

        pltpu.sync_copy(zeros_h.at[pl.ds(s * rpt, rpt)],
                        acc.at[pl.ds(s * rpt, rpt)])
        if rem:
            @pl.when(s == 0)
            def _():
                pltpu.sync_copy(zeros_h.at[pl.ds(rem_base, rem)],
                                acc.at[pl.ds(rem_base, rem)])
        plsc.subcore_barrier()

        wait_gather(0)
        scale_rows(0, 0)
        issue_stage(1, 1)
        wait_dst(0)
        issue_scatter(0)

        assert (nchunk - 1) % 2 == 0

        def pair(T, carry):
            for b in (1, 0):
                o = 1 - b
                t = 2 * T + 1 + (1 - b)
                wait_gather(b)
                scale_rows(t, b)
                wait_scatter(o)

                @pl.when(t + 1 < nchunk)
                def _():
                    issue_stage(t + 1, o)

                wait_dst(b)
                issue_scatter(b)
            return carry

        lax.fori_loop(0, (nchunk - 1) // 2, pair, 0)

        wait_scatter(0)
        plsc.subcore_barrier()
        pltpu.sync_copy(acc.at[pl.ds(s * rpt, rpt)],
                        out_h.at[c, pl.ds(s * rpt, rpt)])
        if rem:
            @pl.when(s == 0)
            def _():
                pltpu.sync_copy(acc.at[pl.ds(rem_base, rem)],
                                out_h.at[c, pl.ds(rem_base, rem)])

    return k(feat, src, dst, w, zeros)


def _tc_epilogue(feat, part0, part1, W0, b0, W1p, b1, gamma, beta):
    n, d_out = feat.shape[0], W0.shape[1]

    def body(feat_r, p0_r, p1_r, w0_r, b0_r, w1_r, b1_r, g_r, be_r, out_r):
        hop1 = p0_r[...] + p1_r[...]
        p0 = jnp.maximum(
            jnp.dot(feat_r[...], w0_r[...], preferred_element_type=jnp.float32),
            0.0) + b0_r[...]
        p1 = jnp.maximum(
            jnp.dot(hop1, w1_r[...], preferred_element_type=jnp.float32),
            0.0) + b1_r[...]
        y = p0 + p1
        mean = jnp.mean(y, axis=0, keepdims=True)
        var = jnp.mean((y - mean) * (y - mean), axis=0, keepdims=True)
        inv = lax.rsqrt(var + EPS) * g_r[...]
        out_r[...] = (y - mean) * inv + be_r[...]

    return pl.pallas_call(
        body,
        out_shape=jax.ShapeDtypeStruct((n, d_out), jnp.float32),
    )(feat, part0, part1, W0, b0.reshape(1, -1), W1p, b1.reshape(1, -1),
      gamma.reshape(1, -1), beta.reshape(1, -1))


def kernel(feat, edge_index, edge_weight, W0, b0, W1, b1, gamma, beta):
    n, d = feat.shape
    dst = edge_index[0]
    src = edge_index[1]
    zeros = jnp.zeros((n, d), jnp.float32)
    w_pad = jnp.concatenate([edge_weight, jnp.zeros((NW * L,), jnp.float32)])
    parts = _sc_spmm_partials(feat, src, dst, w_pad, zeros)
    return _tc_epilogue(feat, parts[0], parts[1], W0, b0, W1, b1,
                        gamma, beta)


# speedup vs baseline: 1.4882x; 1.4882x over previous
import functools

import jax
import jax.numpy as jnp
import numpy as np
from jax import lax
from jax.experimental import pallas as pl
from jax.experimental.pallas import tpu as pltpu
from jax.experimental.pallas import tpu_sc as plsc

EPS = 1e-5

NC = 2
NS = 16
L = 16
NW = NC * NS

K = 80


def _col_perm(d):
    p = []
    for j in range(0, d, 2 * L):
        p.extend(j + 2 * np.arange(L))
        p.extend(j + 2 * np.arange(L) + 1)
    return np.asarray(p, np.int32)


def _sc_spmm_partials(feat, src, dst, w, zeros):
    n, d = feat.shape
    e = src.shape[0]
    ept = e // NW
    nchunk = ept // K

    rpt = (n // NS) // 8 * 8
    rem = n - NS * rpt
    rem_base = NS * rpt

    mesh = plsc.VectorSubcoreMesh(core_axis_name="c", subcore_axis_name="s")

    @functools.partial(
        pl.kernel,
        mesh=mesh,
        out_type=jax.ShapeDtypeStruct((NC, n, d), jnp.float32),
        scratch_types=[
            pltpu.VMEM((ept,), jnp.int32),
            pltpu.VMEM((K + L,), jnp.float32),
            pltpu.VMEM((K + L,), jnp.float32),
            pltpu.VMEM((K + L,), jnp.float32),
            pltpu.VMEM((K,), jnp.int32),
            pltpu.VMEM((K,), jnp.int32),
            pltpu.VMEM((K,), jnp.int32),
            pltpu.VMEM((K, d), jnp.float32),
            pltpu.VMEM((K, d), jnp.float32),
            pltpu.VMEM((K, d), jnp.float32),
            pltpu.VMEM_SHARED((n, d), jnp.float32),
            pltpu.SemaphoreType.DMA,
            pltpu.SemaphoreType.DMA,
            pltpu.SemaphoreType.DMA,
            pltpu.SemaphoreType.DMA,
            pltpu.SemaphoreType.DMA,
            pltpu.SemaphoreType.DMA,
            pltpu.SemaphoreType.DMA,
            pltpu.SemaphoreType.DMA,
            pltpu.SemaphoreType.DMA,
            pltpu.SemaphoreType.DMA,
            pltpu.SemaphoreType.DMA,
            pltpu.SemaphoreType.DMA,
        ],
    )
    def k(feat_h, src_h, dst_h, w_h, zeros_h, out_h,
          src_v, w0, w1, w2, d0, d1, d2, r0, r1, r2, acc,
          gsem0, gsem1, gsem2, wsem0, wsem1, wsem2,
          dsem0, dsem1, dsem2, ssem0, ssem1, ssem2):
        c = lax.axis_index("c")
        s = lax.axis_index("s")
        wid = s * NC + c
        rbf = (r0, r1, r2)
        wb = (w0, w1, w2)
        dstb = (d0, d1, d2)
        gsem = (gsem0, gsem1, gsem2)
        wsem = (wsem0, wsem1, wsem2)
        dsem = (dsem0, dsem1, dsem2)
        ssem = (ssem0, ssem1, ssem2)

        def issue_stage(t, b):
            base = wid * ept + t * K
            pltpu.async_copy(dst_h.at[pl.ds(base, K)], dstb[b], dsem[b])
            pltpu.async_copy(w_h.at[pl.ds(base, K + L)], wb[b], wsem[b])
            pltpu.async_copy(feat_h.at[src_v.at[pl.ds(t * K, K)]],
                             rbf[b], gsem[b])

        def wait_gather(b):
            pltpu.make_async_copy(feat_h.at[pl.ds(0, K)], rbf[b],
                                  gsem[b]).wait()

        def wait_w(b):
            pltpu.make_async_copy(w_h.at[pl.ds(0, K + L)], wb[b],
                                  wsem[b]).wait()

        def wait_dst(b):
            pltpu.make_async_copy(dst_h.at[pl.ds(0, K)], dstb[b],
                                  dsem[b]).wait()

        def wait_scatter(b):
            pltpu.make_async_copy(zeros_h.at[pl.ds(0, K)], rbf[b],
                                  ssem[b]).wait()

        def issue_scatter(b):
            pltpu.async_copy(rbf[b], acc.at[dstb[b]], ssem[b], add=True)

        def scale_rows(b):
            def g_body(g, carry):
                wvec = wb[b][pl.ds(g * L, L)]
                for ii in range(L):
                    i = g * L + ii
                    ws = wvec[ii]
                    for j in range(d // L):
                        rbf[b][i, pl.ds(j * L, L)] = (
                            rbf[b][i, pl.ds(j * L, L)] * ws)
                return carry
            lax.fori_loop(0, K // L, g_body, 0)

        pltpu.sync_copy(src_h.at[pl.ds(wid * ept, ept)], src_v)

        issue_stage(0, 0)
        issue_stage(1, 1)
        issue_stage(2, 2)

        pltpu.sync_copy(zeros_h.at[pl.ds(s * rpt, rpt)],
                        acc.at[pl.ds(s * rpt, rpt)])
        if rem:
            @pl.when(s == 0)
            def _():
                pltpu.sync_copy(zeros_h.at[pl.ds(rem_base, rem)],
                                acc.at[pl.ds(rem_base, rem)])
        plsc.subcore_barrier()

        for t0 in (0, 1):
            wait_gather(t0)
            wait_w(t0)
            scale_rows(t0)
            wait_dst(t0)
            issue_scatter(t0)

        assert (nchunk - 2) % 3 == 0

        def triple(T, carry):
            for j in range(3):
                t = 3 * T + 2 + j
                b = (2 + j) % 3
                wait_scatter(j)

                @pl.when(t + 1 < nchunk)
                def _():
                    issue_stage(t + 1, j)

                wait_gather(b)
                wait_w(b)
                scale_rows(b)
                wait_dst(b)
                issue_scatter(b)
            return carry

        lax.fori_loop(0, (nchunk - 2) // 3, triple, 0)

        wait_scatter((nchunk - 2) % 3)
        wait_scatter((nchunk - 1) % 3)
        plsc.subcore_barrier()
        pltpu.sync_copy(acc.at[pl.ds(s * rpt, rpt)],
                        out_h.at[c, pl.ds(s * rpt, rpt)])
        if rem:
            @pl.when(s == 0)
            def _():
                pltpu.sync_copy(acc.at[pl.ds(rem_base, rem)],
                                out_h.at[c, pl.ds(rem_base, rem)])

    return k(feat, src, dst, w, zeros)


def _tc_epilogue(feat, part0, part1, W0, b0, W1p, b1, gamma, beta):
    n, d_out = feat.shape[0], W0.shape[1]

    def body(feat_r, p0_r, p1_r, w0_r, b0_r, w1_r, b1_r, g_r, be_r, out_r):
        hop1 = p0_r[...] + p1_r[...]
        p0 = jnp.maximum(
            jnp.dot(feat_r[...], w0_r[...], preferred_element_type=jnp.float32),
            0.0) + b0_r[...]
        p1 = jnp.maximum(
            jnp.dot(hop1, w1_r[...], preferred_element_type=jnp.float32),
            0.0) + b1_r[...]
        y = p0 + p1
        mean = jnp.mean(y, axis=0, keepdims=True)
        var = jnp.mean((y - mean) * (y - mean), axis=0, keepdims=True)
        inv = lax.rsqrt(var + EPS) * g_r[...]
        out_r[...] = (y - mean) * inv + be_r[...]

    return pl.pallas_call(
        body,
        out_shape=jax.ShapeDtypeStruct((n, d_out), jnp.float32),
    )(feat, part0, part1, W0, b0.reshape(1, -1), W1p, b1.reshape(1, -1),
      gamma.reshape(1, -1), beta.reshape(1, -1))


def kernel(feat, edge_index, edge_weight, W0, b0, W1, b1, gamma, beta):
    n, d = feat.shape
    dst = edge_index[0]
    src = edge_index[1]
    zeros = jnp.zeros((n, d), jnp.float32)
    w_pad = jnp.concatenate([edge_weight, jnp.zeros((NW * L,), jnp.float32)])
    parts = _sc_spmm_partials(feat, src, dst, w_pad, zeros)
    return _tc_epilogue(feat, parts[0], parts[1], W0, b0, W1, b1,
                        gamma, beta)


# speedup vs baseline: 1.4998x; 1.0078x over previous
import functools

import jax
import jax.numpy as jnp
import numpy as np
from jax import lax
from jax.experimental import pallas as pl
from jax.experimental.pallas import tpu as pltpu
from jax.experimental.pallas import tpu_sc as plsc

EPS = 1e-5

NC = 2
NS = 16
L = 16
NW = NC * NS

K = 80


def _col_perm(d):
    p = []
    for j in range(0, d, 2 * L):
        p.extend(j + 2 * np.arange(L))
        p.extend(j + 2 * np.arange(L) + 1)
    return np.asarray(p, np.int32)


def _sc_spmm_partials(feat, src, dst, w, zeros):
    n, d = feat.shape
    e = src.shape[0]
    ept = e // NW
    nchunk = ept // K

    rpt = (n // NS) // 8 * 8
    rem = n - NS * rpt
    rem_base = NS * rpt

    mesh = plsc.VectorSubcoreMesh(core_axis_name="c", subcore_axis_name="s")

    @functools.partial(
        pl.kernel,
        mesh=mesh,
        out_type=jax.ShapeDtypeStruct((NC, n, d), jnp.float32),
        scratch_types=[
            pltpu.VMEM((ept,), jnp.int32),
            pltpu.VMEM((K,), jnp.float32),
            pltpu.VMEM((K,), jnp.float32),
            pltpu.VMEM((K,), jnp.float32),
            pltpu.VMEM((K,), jnp.int32),
            pltpu.VMEM((K,), jnp.int32),
            pltpu.VMEM((K,), jnp.int32),
            pltpu.VMEM((K, d), jnp.float32),
            pltpu.VMEM((K, d), jnp.float32),
            pltpu.VMEM((K, d), jnp.float32),
            pltpu.VMEM_SHARED((n, d), jnp.float32),
            pltpu.SemaphoreType.DMA,
            pltpu.SemaphoreType.DMA,
            pltpu.SemaphoreType.DMA,
            pltpu.SemaphoreType.DMA,
            pltpu.SemaphoreType.DMA,
            pltpu.SemaphoreType.DMA,
            pltpu.SemaphoreType.DMA,
            pltpu.SemaphoreType.DMA,
            pltpu.SemaphoreType.DMA,
            pltpu.SemaphoreType.DMA,
            pltpu.SemaphoreType.DMA,
            pltpu.SemaphoreType.DMA,
        ],
    )
    def k(feat_h, src_h, dst_h, w_h, zeros_h, out_h,
          src_v, w0, w1, w2, d0, d1, d2, r0, r1, r2, acc,
          gsem0, gsem1, gsem2, wsem0, wsem1, wsem2,
          dsem0, dsem1, dsem2, ssem0, ssem1, ssem2):
        c = lax.axis_index("c")
        s = lax.axis_index("s")
        wid = s * NC + c
        rbf = (r0, r1, r2)
        wb = (w0, w1, w2)
        dstb = (d0, d1, d2)
        gsem = (gsem0, gsem1, gsem2)
        wsem = (wsem0, wsem1, wsem2)
        dsem = (dsem0, dsem1, dsem2)
        ssem = (ssem0, ssem1, ssem2)

        def issue_stage(t, b):
            base = wid * ept + t * K
            pltpu.async_copy(dst_h.at[pl.ds(base, K)], dstb[b], dsem[b])
            pltpu.async_copy(w_h.at[pl.ds(base, K)], wb[b], wsem[b])
            pltpu.async_copy(feat_h.at[src_v.at[pl.ds(t * K, K)]],
                             rbf[b], gsem[b])

        def wait_gather(b):
            pltpu.make_async_copy(feat_h.at[pl.ds(0, K)], rbf[b],
                                  gsem[b]).wait()

        def wait_w(b):
            pltpu.make_async_copy(w_h.at[pl.ds(0, K)], wb[b],
                                  wsem[b]).wait()

        def wait_dst(b):
            pltpu.make_async_copy(dst_h.at[pl.ds(0, K)], dstb[b],
                                  dsem[b]).wait()

        def wait_scatter(b):
            pltpu.make_async_copy(zeros_h.at[pl.ds(0, K)], rbf[b],
                                  ssem[b]).wait()

        def issue_scatter(b):
            pltpu.async_copy(rbf[b], acc.at[dstb[b]], ssem[b], add=True)

        def scale_rows(b):
            def g_body(g, carry):
                wvec = wb[b][pl.ds(g * L, L)]
                for ii in range(L):
                    i = g * L + ii
                    ws = wvec[ii]
                    for j in range(d // L):
                        rbf[b][i, pl.ds(j * L, L)] = (
                            rbf[b][i, pl.ds(j * L, L)] * ws)
                return carry
            lax.fori_loop(0, K // L, g_body, 0)

        pltpu.sync_copy(src_h.at[pl.ds(wid * ept, ept)], src_v)

        issue_stage(0, 0)
        issue_stage(1, 1)
        issue_stage(2, 2)

        pltpu.sync_copy(zeros_h.at[pl.ds(s * rpt, rpt)],
                        acc.at[pl.ds(s * rpt, rpt)])
        if rem:
            @pl.when(s == 0)
            def _():
                pltpu.sync_copy(zeros_h.at[pl.ds(rem_base, rem)],
                                acc.at[pl.ds(rem_base, rem)])
        plsc.subcore_barrier()

        for t0 in (0, 1):
            wait_gather(t0)
            wait_w(t0)
            scale_rows(t0)
            wait_dst(t0)
            issue_scatter(t0)

        assert (nchunk - 2) % 3 == 0

        def triple(T, carry):
            for j in range(3):
                t = 3 * T + 2 + j
                b = (2 + j) % 3
                wait_scatter(j)

                @pl.when(t + 1 < nchunk)
                def _():
                    issue_stage(t + 1, j)

                wait_gather(b)
                wait_w(b)
                scale_rows(b)
                wait_dst(b)
                issue_scatter(b)
            return carry

        lax.fori_loop(0, (nchunk - 2) // 3, triple, 0)

        wait_scatter((nchunk - 2) % 3)
        wait_scatter((nchunk - 1) % 3)
        plsc.subcore_barrier()
        pltpu.sync_copy(acc.at[pl.ds(s * rpt, rpt)],
                        out_h.at[c, pl.ds(s * rpt, rpt)])
        if rem:
            @pl.when(s == 0)
            def _():
                pltpu.sync_copy(acc.at[pl.ds(rem_base, rem)],
                                out_h.at[c, pl.ds(rem_base, rem)])

    return k(feat, src, dst, w, zeros)


def _tc_epilogue(feat, part0, part1, W0, b0, W1p, b1, gamma, beta):
    n, d_out = feat.shape[0], W0.shape[1]

    def body(feat_r, p0_r, p1_r, w0_r, b0_r, w1_r, b1_r, g_r, be_r, out_r):
        hop1 = p0_r[...] + p1_r[...]
        p0 = jnp.maximum(
            jnp.dot(feat_r[...], w0_r[...], preferred_element_type=jnp.float32),
            0.0) + b0_r[...]
        p1 = jnp.maximum(
            jnp.dot(hop1, w1_r[...], preferred_element_type=jnp.float32),
            0.0) + b1_r[...]
        y = p0 + p1
        mean = jnp.mean(y, axis=0, keepdims=True)
        var = jnp.mean((y - mean) * (y - mean), axis=0, keepdims=True)
        inv = lax.rsqrt(var + EPS) * g_r[...]
        out_r[...] = (y - mean) * inv + be_r[...]

    return pl.pallas_call(
        body,
        out_shape=jax.ShapeDtypeStruct((n, d_out), jnp.float32),
    )(feat, part0, part1, W0, b0.reshape(1, -1), W1p, b1.reshape(1, -1),
      gamma.reshape(1, -1), beta.reshape(1, -1))


def kernel(feat, edge_index, edge_weight, W0, b0, W1, b1, gamma, beta):
    n, d = feat.shape
    dst = edge_index[0]
    src = edge_index[1]
    zeros = jnp.zeros((n, d), jnp.float32)
    parts = _sc_spmm_partials(feat, src, dst, edge_weight, zeros)
    return _tc_epilogue(feat, parts[0], parts[1], W0, b0, W1, b1,
                        gamma, beta)


# speedup vs baseline: 1.5013x; 1.0010x over previous
import functools

import jax
import jax.numpy as jnp
from jax import lax
from jax.experimental import pallas as pl
from jax.experimental.pallas import tpu as pltpu
from jax.experimental.pallas import tpu_sc as plsc

EPS = 1e-5

NC = 2
NS = 16
L = 16
NW = NC * NS

K = 80


def _sc_spmm_partials(feat, src, dst, w, zeros):
    n, d = feat.shape
    e = src.shape[0]
    ept = e // NW
    nchunk = ept // K

    rpt = (n // NS) // 8 * 8
    rem = n - NS * rpt
    rem_base = NS * rpt

    mesh = plsc.VectorSubcoreMesh(core_axis_name="c", subcore_axis_name="s")

    @functools.partial(
        pl.kernel,
        mesh=mesh,
        out_type=jax.ShapeDtypeStruct((NC, n, d), jnp.float32),
        scratch_types=[
            pltpu.VMEM((ept,), jnp.int32),
            pltpu.VMEM((K,), jnp.float32),
            pltpu.VMEM((K,), jnp.float32),
            pltpu.VMEM((K,), jnp.float32),
            pltpu.VMEM((K,), jnp.int32),
            pltpu.VMEM((K,), jnp.int32),
            pltpu.VMEM((K,), jnp.int32),
            pltpu.VMEM((K, d), jnp.float32),
            pltpu.VMEM((K, d), jnp.float32),
            pltpu.VMEM((K, d), jnp.float32),
            pltpu.VMEM_SHARED((n, d), jnp.float32),
            pltpu.SemaphoreType.DMA,
            pltpu.SemaphoreType.DMA,
            pltpu.SemaphoreType.DMA,
            pltpu.SemaphoreType.DMA,
            pltpu.SemaphoreType.DMA,
            pltpu.SemaphoreType.DMA,
            pltpu.SemaphoreType.DMA,
            pltpu.SemaphoreType.DMA,
            pltpu.SemaphoreType.DMA,
            pltpu.SemaphoreType.DMA,
            pltpu.SemaphoreType.DMA,
            pltpu.SemaphoreType.DMA,
        ],
    )
    def k(feat_h, src_h, dst_h, w_h, zeros_h, out_h,
          src_v, w0, w1, w2, d0, d1, d2, r0, r1, r2, acc,
          gsem0, gsem1, gsem2, wsem0, wsem1, wsem2,
          dsem0, dsem1, dsem2, ssem0, ssem1, ssem2):
        c = lax.axis_index("c")
        s = lax.axis_index("s")
        wid = s * NC + c
        rbf = (r0, r1, r2)
        wb = (w0, w1, w2)
        dstb = (d0, d1, d2)
        gsem = (gsem0, gsem1, gsem2)
        wsem = (wsem0, wsem1, wsem2)
        dsem = (dsem0, dsem1, dsem2)
        ssem = (ssem0, ssem1, ssem2)

        def issue_stage(t, b):
            base = wid * ept + t * K
            pltpu.async_copy(dst_h.at[pl.ds(base, K)], dstb[b], dsem[b])
            pltpu.async_copy(w_h.at[pl.ds(base, K)], wb[b], wsem[b])
            pltpu.async_copy(feat_h.at[src_v.at[pl.ds(t * K, K)]],
                             rbf[b], gsem[b])

        def wait_gather(b):
            pltpu.make_async_copy(feat_h.at[pl.ds(0, K)], rbf[b],
                                  gsem[b]).wait()

        def wait_w(b):
            pltpu.make_async_copy(w_h.at[pl.ds(0, K)], wb[b],
                                  wsem[b]).wait()

        def wait_dst(b):
            pltpu.make_async_copy(dst_h.at[pl.ds(0, K)], dstb[b],
                                  dsem[b]).wait()

        def wait_scatter(b):
            pltpu.make_async_copy(zeros_h.at[pl.ds(0, K)], rbf[b],
                                  ssem[b]).wait()

        def issue_scatter(b):
            pltpu.async_copy(rbf[b], acc.at[dstb[b]], ssem[b], add=True)

        def scale_rows(b):
            def g_body(g, carry):
                wvec = wb[b][pl.ds(g * L, L)]
                for ii in range(L):
                    i = g * L + ii
                    ws = wvec[ii]
                    for j in range(d // L):
                        rbf[b][i, pl.ds(j * L, L)] = (
                            rbf[b][i, pl.ds(j * L, L)] * ws)
                return carry
            lax.fori_loop(0, K // L, g_body, 0)

        pltpu.sync_copy(src_h.at[pl.ds(wid * ept, ept)], src_v)

        issue_stage(0, 0)
        issue_stage(1, 1)
        issue_stage(2, 2)

        pltpu.sync_copy(zeros_h.at[pl.ds(s * rpt, rpt)],
                        acc.at[pl.ds(s * rpt, rpt)])
        if rem:
            @pl.when(s == 0)
            def _():
                pltpu.sync_copy(zeros_h.at[pl.ds(rem_base, rem)],
                                acc.at[pl.ds(rem_base, rem)])
        plsc.subcore_barrier()

        for t0 in (0, 1):
            wait_gather(t0)
            wait_w(t0)
            scale_rows(t0)
            wait_dst(t0)
            issue_scatter(t0)

        assert (nchunk - 2) % 3 == 0

        def triple(T, carry):
            for j in range(3):
                t = 3 * T + 2 + j
                b = (2 + j) % 3
                wait_scatter(j)

                @pl.when(t + 1 < nchunk)
                def _():
                    issue_stage(t + 1, j)

                wait_gather(b)
                wait_w(b)
                scale_rows(b)
                wait_dst(b)
                issue_scatter(b)
            return carry

        lax.fori_loop(0, (nchunk - 2) // 3, triple, 0)

        wait_scatter((nchunk - 2) % 3)
        wait_scatter((nchunk - 1) % 3)
        plsc.subcore_barrier()
        pltpu.sync_copy(acc.at[pl.ds(s * rpt, rpt)],
                        out_h.at[c, pl.ds(s * rpt, rpt)])
        if rem:
            @pl.when(s == 0)
            def _():
                pltpu.sync_copy(acc.at[pl.ds(rem_base, rem)],
                                out_h.at[c, pl.ds(rem_base, rem)])

    return k(feat, src, dst, w, zeros)


def _tc_epilogue(feat, part0, part1, W0, b0, W1, b1, gamma, beta):
    n, d_out = feat.shape[0], W0.shape[1]

    def body(feat_r, p0_r, p1_r, w0_r, b0_r, w1_r, b1_r, g_r, be_r, out_r):
        hop1 = p0_r[...] + p1_r[...]
        p0 = jnp.maximum(
            jnp.dot(feat_r[...], w0_r[...], preferred_element_type=jnp.float32),
            0.0) + b0_r[...]
        p1 = jnp.maximum(
            jnp.dot(hop1, w1_r[...], preferred_element_type=jnp.float32),
            0.0) + b1_r[...]
        y = p0 + p1
        mean = jnp.mean(y, axis=0, keepdims=True)
        var = jnp.mean((y - mean) * (y - mean), axis=0, keepdims=True)
        inv = lax.rsqrt(var + EPS) * g_r[...]
        out_r[...] = (y - mean) * inv + be_r[...]

    return pl.pallas_call(
        body,
        out_shape=jax.ShapeDtypeStruct((n, d_out), jnp.float32),
    )(feat, part0, part1, W0, b0.reshape(1, -1), W1, b1.reshape(1, -1),
      gamma.reshape(1, -1), beta.reshape(1, -1))


def kernel(feat, edge_index, edge_weight, W0, b0, W1, b1, gamma, beta):
    n, d = feat.shape
    dst = edge_index[0]
    src = edge_index[1]
    zeros = jnp.zeros((n, d), jnp.float32)
    parts = _sc_spmm_partials(feat, src, dst, edge_weight, zeros)
    return _tc_epilogue(feat, parts[0], parts[1], W0, b0, W1, b1,
                        gamma, beta)
